# baseline (speedup 1.0000x reference)
import jax
import jax.numpy as jnp
from jax.experimental import pallas as pl
from jax.experimental.pallas import tpu as pltpu

P = 26
K = 64
B = 16384
N = B * K
LAM = 0.1

C = 65536
GRID = N // C


def _copy_kernel(x_ref, out_ref):
    out_ref[...] = x_ref[...]


@jax.jit
def kernel(partition_outputs, pos_embedding):
    xf = partition_outputs.reshape(P, N)
    out = pl.pallas_call(
        _copy_kernel,
        grid=(GRID,),
        in_specs=[pl.BlockSpec((P, C), lambda j: (0, j))],
        out_specs=pl.BlockSpec((P, C), lambda j: (0, j)),
        out_shape=jax.ShapeDtypeStruct((P, N), jnp.float32),
    )(xf)
    return out.reshape(P, B, K), jnp.float32(0.0)


# speedup vs baseline: 1.9254x; 1.9254x over previous
import jax
import jax.numpy as jnp
from jax.experimental import pallas as pl
from jax.experimental.pallas import tpu as pltpu

P = 26
K = 64
B = 16384
LAM = 0.1

S = 8
BB = 256
GRID = B // BB
PS = P * S
CW = (BB // S) * K
NBUF = 4
Q = 13
PPQ = P // Q


def _in_copy(x_hbm, ibuf, isem, blk, slot):
    for q in range(Q):
        yield pltpu.make_async_copy(
            x_hbm.at[pl.ds(q * PPQ, PPQ), pl.ds(blk * BB, BB), :],
            ibuf.at[slot, pl.ds(q * PPQ, PPQ)],
            isem.at[slot, q])


def _out_copy(obuf, out_hbm, osem, blk, slot):
    for q in range(Q):
        yield pltpu.make_async_copy(
            obuf.at[slot, pl.ds(q * PPQ, PPQ)],
            out_hbm.at[pl.ds(q * PPQ, PPQ), pl.ds(blk * BB, BB), :],
            osem.at[slot, q])


def _fused_kernel(x_hbm, pos_ref, out_hbm, loss_ref,
                  ibuf, obuf, acc_ref, isem, osem):
    j = pl.program_id(0)
    slot = jax.lax.rem(j, NBUF)

    @pl.when(j == 0)
    def _prologue():
        acc_ref[...] = jnp.zeros_like(acc_ref)
        for c in _in_copy(x_hbm, ibuf, isem, 0, 0):
            c.start()
        for c in _in_copy(x_hbm, ibuf, isem, 1, 1):
            c.start()

    @pl.when(j + 2 < GRID)
    def _prefetch():
        for c in _in_copy(x_hbm, ibuf, isem, j + 2, jax.lax.rem(j + 2, NBUF)):
            c.start()

    for c in _in_copy(x_hbm, ibuf, isem, j, slot):
        c.wait()

    x = ibuf[slot]

    z = x.reshape(PS, BB // S, K).astype(jnp.bfloat16).reshape(PS, CW)
    acc_ref[...] += jax.lax.dot_general(
        z, z, dimension_numbers=(((1,), (1,)), ((), ())),
        preferred_element_type=jnp.float32)

    @pl.when(j >= NBUF)
    def _drain():
        for c in _out_copy(obuf, out_hbm, osem, j - NBUF, slot):
            c.wait()

    obuf[slot] = x + pos_ref[...]

    for c in _out_copy(obuf, out_hbm, osem, j, slot):
        c.start()

    @pl.when(j == GRID - 1)
    def _epilogue():
        for d in range(NBUF):
            blk = GRID - NBUF + d
            for c in _out_copy(obuf, out_hbm, osem, blk, blk % NBUF):
                c.wait()

        zz = acc_ref[...]
        ra = jax.lax.broadcasted_iota(jnp.int32, (PS, PS), 0)
        rb = jax.lax.broadcasted_iota(jnp.int32, (PS, PS), 1)
        zz = jnp.where(ra % S == rb % S, zz, 0.0)
        pa = jax.lax.broadcasted_iota(jnp.int32, (P, PS), 0)
        pb = jax.lax.broadcasted_iota(jnp.int32, (P, PS), 1)
        sel = (pa == pb // S).astype(jnp.float32)
        t = jax.lax.dot_general(
            sel, zz, dimension_numbers=(((1,), (0,)), ((), ())),
            preferred_element_type=jnp.float32)
        g = jax.lax.dot_general(
            t, sel, dimension_numbers=(((1,), (1,)), ((), ())),
            preferred_element_type=jnp.float32)
        ri = jax.lax.broadcasted_iota(jnp.int32, (P, P), 0)
        ci = jax.lax.broadcasted_iota(jnp.int32, (P, P), 1)
        eye = ri == ci
        diag_r = jnp.sum(jnp.where(eye, g, 0.0), axis=1, keepdims=True)
        diag_c = jnp.sum(jnp.where(eye, g, 0.0), axis=0, keepdims=True)
        denom = (jnp.sqrt(diag_r) + 1e-8) * (jnp.sqrt(diag_c) + 1e-8)
        gn = g / denom
        off2 = jnp.where(eye, 0.0, gn * gn)
        loss = LAM * jnp.sum(off2) / (P * (P - 1))
        loss_ref[...] = loss.reshape(1, 1)


@jax.jit
def kernel(partition_outputs, pos_embedding):
    pos3 = pos_embedding.reshape(P, 1, K)

    out, loss = pl.pallas_call(
        _fused_kernel,
        grid=(GRID,),
        in_specs=[
            pl.BlockSpec(memory_space=pltpu.MemorySpace.HBM),
            pl.BlockSpec((P, 1, K), lambda j: (0, 0, 0)),
        ],
        out_specs=[
            pl.BlockSpec(memory_space=pltpu.MemorySpace.HBM),
            pl.BlockSpec((1, 1), lambda j: (0, 0)),
        ],
        out_shape=[
            jax.ShapeDtypeStruct((P, B, K), jnp.float32),
            jax.ShapeDtypeStruct((1, 1), jnp.float32),
        ],
        scratch_shapes=[
            pltpu.VMEM((NBUF, P, BB, K), jnp.float32),
            pltpu.VMEM((NBUF, P, BB, K), jnp.float32),
            pltpu.VMEM((PS, PS), jnp.float32),
            pltpu.SemaphoreType.DMA((NBUF, Q)),
            pltpu.SemaphoreType.DMA((NBUF, Q)),
        ],
    )(partition_outputs, pos3)

    return out, loss[0, 0]
